# baseline (speedup 1.0000x reference)
import jax
import jax.numpy as jnp
from jax.experimental import pallas as pl
from jax.experimental.pallas import tpu as pltpu

_BLOCK = 8192
_SUB = 512
_NOUT = 4
_NE = 8


def _top2_block(x, wt):
    logits = jnp.dot(x, wt, preferred_element_type=jnp.float32)
    lane = jax.lax.broadcasted_iota(jnp.int32, logits.shape, 1)
    l1 = jnp.max(logits, axis=-1, keepdims=True)
    i1 = jnp.argmax(logits, axis=-1).astype(jnp.int32)[:, None]
    masked = jnp.where(lane == i1, -jnp.inf, logits)
    l2 = jnp.max(masked, axis=-1, keepdims=True)
    i2 = jnp.argmax(masked, axis=-1).astype(jnp.int32)[:, None]
    t = jnp.exp(l2 - l1)
    w1 = 1.0 / (1.0 + t)
    w2 = t * w1
    idx = jnp.concatenate([i1, i2], axis=1)
    w = jnp.concatenate([w1, w2], axis=1)
    return idx, w


def _gate_body(x_ref, wt_ref, idx_hbm, w_hbm, ibuf, wbuf, isems, wsems):
    j = pl.program_id(0)
    nsub = _BLOCK // _SUB
    total = pl.num_programs(0) * nsub

    def out_copy(g, oslot):
        return (
            pltpu.make_async_copy(
                ibuf.at[oslot], idx_hbm.at[pl.ds(g * _SUB, _SUB), :],
                isems.at[oslot],
            ),
            pltpu.make_async_copy(
                wbuf.at[oslot], w_hbm.at[pl.ds(g * _SUB, _SUB), :],
                wsems.at[oslot],
            ),
        )

    def sub(sb, c):
        g = j * nsub + sb
        oslot = jax.lax.rem(g, _NOUT)
        idx, w = _top2_block(x_ref[pl.ds(sb * _SUB, _SUB), :], wt_ref[...])

        @pl.when(g >= _NOUT)
        def _():
            pic, pwc = out_copy(g - _NOUT, oslot)
            pic.wait()
            pwc.wait()

        ibuf[oslot] = idx
        wbuf[oslot] = w
        ic, wc = out_copy(g, oslot)
        ic.start()
        wc.start()
        return c

    jax.lax.fori_loop(0, nsub, sub, 0)

    @pl.when(j == pl.num_programs(0) - 1)
    def _():
        for k in range(_NOUT):
            g = total - _NOUT + k
            oslot = jax.lax.rem(g, _NOUT)
            ic, wc = out_copy(g, oslot)
            ic.wait()
            wc.wait()


def _route(x, wt):
    n, h = x.shape
    grid = n // _BLOCK
    return pl.pallas_call(
        _gate_body,
        grid=(grid,),
        in_specs=[
            pl.BlockSpec((_BLOCK, h), lambda i: (i, 0)),
            pl.BlockSpec(memory_space=pltpu.VMEM),
        ],
        out_specs=[
            pl.BlockSpec(memory_space=pl.ANY),
            pl.BlockSpec(memory_space=pl.ANY),
        ],
        out_shape=[
            jax.ShapeDtypeStruct((n, 2), jnp.int32),
            jax.ShapeDtypeStruct((n, 2), jnp.float32),
        ],
        scratch_shapes=[
            pltpu.VMEM((_NOUT, _SUB, 2), jnp.int32),
            pltpu.VMEM((_NOUT, _SUB, 2), jnp.float32),
            pltpu.SemaphoreType.DMA((_NOUT,)),
            pltpu.SemaphoreType.DMA((_NOUT,)),
        ],
        compiler_params=pltpu.CompilerParams(
            dimension_semantics=("arbitrary",),
            vmem_limit_bytes=62 * 1024 * 1024,
        ),
    )(x, wt)


@jax.jit
def kernel(hidden_states, weight):
    h = hidden_states.shape[-1]
    x = hidden_states.reshape(-1, h)
    topk_idx, topk_weight = _route(x, weight.T)
    return topk_idx, topk_weight
